# baseline (speedup 1.0000x reference)
import functools

import jax
import jax.numpy as jnp
from jax import lax
from jax.experimental import pallas as pl
from jax.experimental.pallas import tpu as pltpu
from jax.experimental.pallas import tpu_sc as plsc

N = 10000
E = 320000
D = 128
H = 128
O = 40
OP = 48
G = 64
CW = 16

NC = 2
NS = 16
NW = NC * NS
EPW = E // NW
K = 80
NCHUNK = EPW // K
NPAD = 10240
RPS = NPAD // NS
SR = 128


NB = 4
NI = 8
MAIN = (NCHUNK // NI) * NI

_PARAMS = pltpu.CompilerParams(use_tc_tiling_on_sc=False)
_MESH = dict(core_axis_name="c", subcore_axis_name="s")


def _sc_aggregate(width, dtype, with_counts):
    mesh = plsc.VectorSubcoreMesh(**_MESH)

    out_type = [jax.ShapeDtypeStruct((NC, NPAD, width), dtype)]
    scratch = [
        pltpu.VMEM_SHARED((NPAD, width), dtype),
        [pltpu.VMEM((K,), jnp.int32)] * NI,
        [pltpu.VMEM((1, K), jnp.int32)] * NI,
        [pltpu.SemaphoreType.DMA] * NI,
        [pltpu.VMEM((K, width), dtype)] * NB,
        [pltpu.SemaphoreType.DMA] * NB,
    ]
    if with_counts:
        out_type.append(jax.ShapeDtypeStruct((NC, NPAD, CW), jnp.float32))
        scratch += [
            pltpu.VMEM_SHARED((NPAD, CW), jnp.float32),
            pltpu.VMEM((K, CW), jnp.float32),
        ]

    def body(*refs):
        if with_counts:
            (feat_hbm, ei_hbm, zw_hbm, zc_hbm, ones_hbm, s_out, c_out,
             acc, sidx, didx, isem, rows, gsem, cacc, ones_v) = refs
        else:
            (feat_hbm, ei_hbm, zw_hbm, s_out,
             acc, sidx, didx, isem, rows, gsem) = refs

        cid = lax.axis_index("c")
        sid = lax.axis_index("s")
        wid = cid * NS + sid
        ebase = wid * EPW
        r0 = sid * RPS

        def idx_load(j, i):
            off = ebase + j * K
            pltpu.async_copy(ei_hbm.at[pl.ds(off, K)], sidx[i], isem[i])
            pltpu.async_copy(ei_hbm.at[pl.ds(E + off, K)], didx[i].at[0],
                             isem[i])

        def idx_wait(j, i):
            off = ebase + j * K
            pltpu.make_async_copy(ei_hbm.at[pl.ds(off, K)], sidx[i],
                                  isem[i]).wait()
            pltpu.make_async_copy(ei_hbm.at[pl.ds(E + off, K)],
                                  didx[i].at[0], isem[i]).wait()

        def gather_start(j, i, b):
            pltpu.async_copy(feat_hbm.at[sidx[i]], rows[b], gsem[b])

        def gather_wait(i, b):
            pltpu.make_async_copy(feat_hbm.at[sidx[i]], rows[b],
                                  gsem[b]).wait()

        for i in range(NI):
            idx_load(i, i)
        pltpu.sync_copy(zw_hbm, acc.at[pl.ds(r0, RPS)])
        if with_counts:
            pltpu.sync_copy(zc_hbm, cacc.at[pl.ds(r0, RPS)])
            pltpu.sync_copy(ones_hbm, ones_v)
        plsc.subcore_barrier()
        for b in range(NB):
            idx_wait(b, b)
            gather_start(b, b, b)

        def process(j, i, b):
            gather_wait(i, b)
            pltpu.sync_copy(rows[b], acc.at[didx[i].at[0]], add=True)
            if with_counts:
                pltpu.sync_copy(ones_v, cacc.at[didx[i].at[0]], add=True)

            @pl.when(j + NI < NCHUNK)
            def _():
                idx_load(j + NI, i)

            @pl.when(j + NB < NCHUNK)
            def _():
                idx_wait(j + NB, (i + NB) % NI)
                gather_start(j + NB, (i + NB) % NI, b)

        @pl.loop(0, MAIN, step=NI)
        def _(j0):
            for t in range(NI):
                process(j0 + t, t, t % NB)

        for j in range(MAIN, NCHUNK):
            process(j, j % NI, j % NB)

        plsc.subcore_barrier()

        pltpu.sync_copy(acc.at[pl.ds(r0, RPS)],
                        s_out.at[cid, pl.ds(r0, RPS)])
        if with_counts:
            pltpu.sync_copy(cacc.at[pl.ds(r0, RPS)],
                            c_out.at[cid, pl.ds(r0, RPS)])

    return pl.kernel(body, out_type=out_type, mesh=mesh,
                     scratch_types=scratch, compiler_params=_PARAMS)


def _sc_aggr(h, ei, width):
    zw = jnp.zeros((RPS, width), h.dtype)
    return _sc_aggregate(width, h.dtype, False)(h, ei, zw)[0]


def _sc_aggr_counts(h, ei, width):
    zw = jnp.zeros((RPS, width), h.dtype)
    zc = jnp.zeros((RPS, CW), jnp.float32)
    ones = jnp.ones((K, CW), jnp.float32)
    return _sc_aggregate(width, h.dtype, True)(h, ei, zw, zc, ones)


RB = 2000


def _tc_layer1_body(s0, s1, c0, c1, x, wl, wr, b, hb_out, inv_out):
    cnt = c0[0][:, :1] + c1[0][:, :1]
    inv = 1.0 / jnp.maximum(cnt, 1.0)
    aggr = (s0[0].astype(jnp.float32) + s1[0].astype(jnp.float32)) * inv
    acc = jnp.dot(aggr, wl[...], preferred_element_type=jnp.float32)
    acc += jnp.dot(x[...], wr[...], preferred_element_type=jnp.float32)
    h = jnp.maximum(acc + b[...], 0.0)
    hb_out[...] = h.astype(jnp.bfloat16)
    inv_out[...] = inv


def _tc_layer1(s, c, x, wl, wr, b):
    grid = N // RB
    return pl.pallas_call(
        _tc_layer1_body,
        grid=(grid,),
        in_specs=[
            pl.BlockSpec((1, RB, D), lambda i: (0, i, 0)),
            pl.BlockSpec((1, RB, D), lambda i: (1, i, 0)),
            pl.BlockSpec((1, RB, CW), lambda i: (0, i, 0)),
            pl.BlockSpec((1, RB, CW), lambda i: (1, i, 0)),
            pl.BlockSpec((RB, D), lambda i: (i, 0)),
            pl.BlockSpec((D, H), lambda i: (0, 0)),
            pl.BlockSpec((D, H), lambda i: (0, 0)),
            pl.BlockSpec((1, H), lambda i: (0, 0)),
        ],
        out_specs=[
            pl.BlockSpec((RB, H), lambda i: (i, 0)),
            pl.BlockSpec((RB, 1), lambda i: (i, 0)),
        ],
        out_shape=[
            jax.ShapeDtypeStruct((N, H), jnp.bfloat16),
            jax.ShapeDtypeStruct((N, 1), jnp.float32),
        ],
    )(s, s, c, c, x, wl, wr, b)


def _tc_layer2_body(s0, s1, inv, h, wl, wr, b, wl3, wr3, b3,
                    p_out, r_out):
    aggr = (s0[0].astype(jnp.float32) + s1[0].astype(jnp.float32)) \
        * inv[...]
    acc = jnp.dot(aggr, wl[...], preferred_element_type=jnp.float32)
    acc += jnp.dot(h[...].astype(jnp.float32), wr[...],
                   preferred_element_type=jnp.float32)
    h2 = jnp.maximum(acc + b[...], 0.0)
    p = jnp.dot(h2, wl3[...], preferred_element_type=jnp.float32)
    p_out[...] = p.astype(jnp.bfloat16)
    r_out[...] = jnp.dot(h2, wr3[...],
                         preferred_element_type=jnp.float32) + b3[...]


def _tc_layer2(s, inv, h, wl, wr, b, wl3, wr3, b3):
    grid = N // RB
    return pl.pallas_call(
        _tc_layer2_body,
        grid=(grid,),
        in_specs=[
            pl.BlockSpec((1, RB, H), lambda i: (0, i, 0)),
            pl.BlockSpec((1, RB, H), lambda i: (1, i, 0)),
            pl.BlockSpec((RB, 1), lambda i: (i, 0)),
            pl.BlockSpec((RB, H), lambda i: (i, 0)),
            pl.BlockSpec((H, H), lambda i: (0, 0)),
            pl.BlockSpec((H, H), lambda i: (0, 0)),
            pl.BlockSpec((1, H), lambda i: (0, 0)),
            pl.BlockSpec((H, OP), lambda i: (0, 0)),
            pl.BlockSpec((H, OP), lambda i: (0, 0)),
            pl.BlockSpec((1, OP), lambda i: (0, 0)),
        ],
        out_specs=[
            pl.BlockSpec((RB, OP), lambda i: (i, 0)),
            pl.BlockSpec((RB, OP), lambda i: (i, 0)),
        ],
        out_shape=[
            jax.ShapeDtypeStruct((N, OP), jnp.bfloat16),
            jax.ShapeDtypeStruct((N, OP), jnp.float32),
        ],
    )(s, s, inv, h, wl, wr, b, wl3, wr3, b3)


def _tc_final_body(s0, s1, inv, r, batch, out):
    h3 = (s0[0].astype(jnp.float32) + s1[0].astype(jnp.float32)) \
        * inv[...] + r[...]
    gids = lax.broadcasted_iota(jnp.int32, (N, G), 1)
    onehot = (batch[...] == gids).astype(jnp.float32)
    pooled = lax.dot_general(onehot, h3, (((0,), (0,)), ((), ())),
                             preferred_element_type=jnp.float32)
    cntg = jnp.sum(onehot, axis=0).reshape(G, 1)
    pm = pooled / jnp.maximum(cntg, 1.0)
    col = lax.broadcasted_iota(jnp.int32, (G, OP), 1)
    valid = col < O
    z = jnp.where(valid, pm, -1e30)
    z = z - jnp.max(z, axis=1, keepdims=True)
    e = jnp.where(valid, jnp.exp(z), 0.0)
    out[...] = e / jnp.sum(e, axis=1, keepdims=True)


def _tc_final(s, inv, r, batch2d):
    return pl.pallas_call(
        _tc_final_body,
        grid=(1,),
        in_specs=[
            pl.BlockSpec((1, N, OP), lambda i: (0, 0, 0)),
            pl.BlockSpec((1, N, OP), lambda i: (1, 0, 0)),
            pl.BlockSpec((N, 1), lambda i: (0, 0)),
            pl.BlockSpec((N, OP), lambda i: (0, 0)),
            pl.BlockSpec((N, 1), lambda i: (0, 0)),
        ],
        out_specs=pl.BlockSpec((G, OP), lambda i: (0, 0)),
        out_shape=jax.ShapeDtypeStruct((G, OP), jnp.float32),
    )(s, s, inv, r, batch2d)


def kernel(x, edge_index, batch, W1l, W1r, b1, W2l, W2r, b2, W3l, W3r, b3):
    ei = jnp.ravel(edge_index.astype(jnp.int32))

    S1, C = _sc_aggr_counts(x.astype(jnp.bfloat16), ei, D)
    h1b, inv = _tc_layer1(S1, C, x, W1l, W1r, b1.reshape(1, H))

    wl3 = jnp.pad(W3l, ((0, 0), (0, OP - O)))
    wr3 = jnp.pad(W3r, ((0, 0), (0, OP - O)))
    b3p = jnp.pad(b3, (0, OP - O)).reshape(1, OP)
    S2 = _sc_aggr(h1b, ei, H)
    p, r = _tc_layer2(S2, inv, h1b, W2l, W2r, b2.reshape(1, H),
                      wl3, wr3, b3p)
    S3 = _sc_aggr(p, ei, OP)

    out = _tc_final(S3, inv, r, batch.astype(jnp.int32).reshape(N, 1))
    return out[:, :O]


# speedup vs baseline: 1.0148x; 1.0148x over previous
import functools

import jax
import jax.numpy as jnp
from jax import lax
from jax.experimental import pallas as pl
from jax.experimental.pallas import tpu as pltpu
from jax.experimental.pallas import tpu_sc as plsc

N = 10000
E = 320000
D = 128
H = 128
O = 40
OP = 48
G = 64
CW = 16

NC = 2
NS = 16
NW = NC * NS
EPW = E // NW
K = 80
NCHUNK = EPW // K
NPAD = 10240
RPS = NPAD // NS
SR = 128


NB = 4
NI = 8
MAIN = (NCHUNK // NI) * NI

_PARAMS = pltpu.CompilerParams(use_tc_tiling_on_sc=False)
_MESH = dict(core_axis_name="c", subcore_axis_name="s")


def _sc_aggregate(width, dtype, with_counts):
    mesh = plsc.VectorSubcoreMesh(**_MESH)

    out_type = [jax.ShapeDtypeStruct((NC, NPAD, width), dtype)]
    scratch = [
        pltpu.VMEM_SHARED((NPAD, width), dtype),
        [pltpu.VMEM((K,), jnp.int32)] * NI,
        [pltpu.VMEM((1, K), jnp.int32)] * NI,
        [pltpu.SemaphoreType.DMA] * NI,
        [pltpu.VMEM((K, width), dtype)] * NB,
        [pltpu.SemaphoreType.DMA] * NB,
    ]
    if with_counts:
        out_type.append(jax.ShapeDtypeStruct((NC, NPAD, CW), jnp.float32))
        scratch += [
            pltpu.VMEM_SHARED((NPAD, CW), jnp.float32),
            pltpu.VMEM((K, CW), jnp.float32),
        ]

    def body(*refs):
        if with_counts:
            (feat_hbm, ei_hbm, zw_hbm, zc_hbm, ones_hbm, s_out, c_out,
             acc, sidx, didx, isem, rows, gsem, cacc, ones_v) = refs
        else:
            (feat_hbm, ei_hbm, zw_hbm, s_out,
             acc, sidx, didx, isem, rows, gsem) = refs

        cid = lax.axis_index("c")
        sid = lax.axis_index("s")
        wid = cid * NS + sid
        ebase = wid * EPW
        r0 = sid * RPS

        def idx_load(j, i):
            off = ebase + j * K
            pltpu.async_copy(ei_hbm.at[pl.ds(off, K)], sidx[i], isem[i])
            pltpu.async_copy(ei_hbm.at[pl.ds(E + off, K)], didx[i].at[0],
                             isem[i])

        def idx_wait(j, i):
            off = ebase + j * K
            pltpu.make_async_copy(ei_hbm.at[pl.ds(off, K)], sidx[i],
                                  isem[i]).wait()
            pltpu.make_async_copy(ei_hbm.at[pl.ds(E + off, K)],
                                  didx[i].at[0], isem[i]).wait()

        def gather_start(j, i, b):
            pltpu.async_copy(feat_hbm.at[sidx[i]], rows[b], gsem[b])

        def gather_wait(i, b):
            pltpu.make_async_copy(feat_hbm.at[sidx[i]], rows[b],
                                  gsem[b]).wait()

        for i in range(NI):
            idx_load(i, i)
        pltpu.sync_copy(zw_hbm, rows[0])
        for kk in range(RPS // K):
            pltpu.sync_copy(rows[0], acc.at[pl.ds(r0 + kk * K, K)])
        if with_counts:
            pltpu.sync_copy(zc_hbm, ones_v)
            for kk in range(RPS // K):
                pltpu.sync_copy(ones_v, cacc.at[pl.ds(r0 + kk * K, K)])
            pltpu.sync_copy(ones_hbm, ones_v)
        plsc.subcore_barrier()
        for b in range(NB):
            idx_wait(b, b)
            gather_start(b, b, b)

        def process(j, i, b):
            gather_wait(i, b)
            pltpu.sync_copy(rows[b], acc.at[didx[i].at[0]], add=True)
            if with_counts:
                pltpu.sync_copy(ones_v, cacc.at[didx[i].at[0]], add=True)

            @pl.when(j + NI < NCHUNK)
            def _():
                idx_load(j + NI, i)

            @pl.when(j + NB < NCHUNK)
            def _():
                idx_wait(j + NB, (i + NB) % NI)
                gather_start(j + NB, (i + NB) % NI, b)

        @pl.loop(0, MAIN, step=NI)
        def _(j0):
            for t in range(NI):
                process(j0 + t, t, t % NB)

        for j in range(MAIN, NCHUNK):
            process(j, j % NI, j % NB)

        plsc.subcore_barrier()

        pltpu.sync_copy(acc.at[pl.ds(r0, RPS)],
                        s_out.at[cid, pl.ds(r0, RPS)])
        if with_counts:
            pltpu.sync_copy(cacc.at[pl.ds(r0, RPS)],
                            c_out.at[cid, pl.ds(r0, RPS)])

    return pl.kernel(body, out_type=out_type, mesh=mesh,
                     scratch_types=scratch, compiler_params=_PARAMS)


def _sc_aggr(h, ei, width):
    zw = jnp.zeros((K, width), h.dtype)
    return _sc_aggregate(width, h.dtype, False)(h, ei, zw)[0]


def _sc_aggr_counts(h, ei, width):
    zw = jnp.zeros((K, width), h.dtype)
    zc = jnp.zeros((K, CW), jnp.float32)
    ones = jnp.ones((K, CW), jnp.float32)
    return _sc_aggregate(width, h.dtype, True)(h, ei, zw, zc, ones)


RB = 2000


def _tc_layer1_body(s0, s1, c0, c1, x, wl, wr, b, hb_out, inv_out):
    cnt = c0[0][:, :1] + c1[0][:, :1]
    inv = 1.0 / jnp.maximum(cnt, 1.0)
    aggr = (s0[0].astype(jnp.float32) + s1[0].astype(jnp.float32)) * inv
    acc = jnp.dot(aggr, wl[...], preferred_element_type=jnp.float32)
    acc += jnp.dot(x[...], wr[...], preferred_element_type=jnp.float32)
    h = jnp.maximum(acc + b[...], 0.0)
    hb_out[...] = h.astype(jnp.bfloat16)
    inv_out[...] = inv


def _tc_layer1(s, c, x, wl, wr, b):
    grid = N // RB
    return pl.pallas_call(
        _tc_layer1_body,
        grid=(grid,),
        in_specs=[
            pl.BlockSpec((1, RB, D), lambda i: (0, i, 0)),
            pl.BlockSpec((1, RB, D), lambda i: (1, i, 0)),
            pl.BlockSpec((1, RB, CW), lambda i: (0, i, 0)),
            pl.BlockSpec((1, RB, CW), lambda i: (1, i, 0)),
            pl.BlockSpec((RB, D), lambda i: (i, 0)),
            pl.BlockSpec((D, H), lambda i: (0, 0)),
            pl.BlockSpec((D, H), lambda i: (0, 0)),
            pl.BlockSpec((1, H), lambda i: (0, 0)),
        ],
        out_specs=[
            pl.BlockSpec((RB, H), lambda i: (i, 0)),
            pl.BlockSpec((RB, 1), lambda i: (i, 0)),
        ],
        out_shape=[
            jax.ShapeDtypeStruct((N, H), jnp.bfloat16),
            jax.ShapeDtypeStruct((N, 1), jnp.float32),
        ],
    )(s, s, c, c, x, wl, wr, b)


def _tc_layer2_body(s0, s1, inv, h, wl, wr, b, wl3, wr3, b3,
                    p_out, r_out):
    aggr = (s0[0].astype(jnp.float32) + s1[0].astype(jnp.float32)) \
        * inv[...]
    acc = jnp.dot(aggr, wl[...], preferred_element_type=jnp.float32)
    acc += jnp.dot(h[...].astype(jnp.float32), wr[...],
                   preferred_element_type=jnp.float32)
    h2 = jnp.maximum(acc + b[...], 0.0)
    p = jnp.dot(h2, wl3[...], preferred_element_type=jnp.float32)
    p_out[...] = p.astype(jnp.bfloat16)
    r_out[...] = jnp.dot(h2, wr3[...],
                         preferred_element_type=jnp.float32) + b3[...]


def _tc_layer2(s, inv, h, wl, wr, b, wl3, wr3, b3):
    grid = N // RB
    return pl.pallas_call(
        _tc_layer2_body,
        grid=(grid,),
        in_specs=[
            pl.BlockSpec((1, RB, H), lambda i: (0, i, 0)),
            pl.BlockSpec((1, RB, H), lambda i: (1, i, 0)),
            pl.BlockSpec((RB, 1), lambda i: (i, 0)),
            pl.BlockSpec((RB, H), lambda i: (i, 0)),
            pl.BlockSpec((H, H), lambda i: (0, 0)),
            pl.BlockSpec((H, H), lambda i: (0, 0)),
            pl.BlockSpec((1, H), lambda i: (0, 0)),
            pl.BlockSpec((H, OP), lambda i: (0, 0)),
            pl.BlockSpec((H, OP), lambda i: (0, 0)),
            pl.BlockSpec((1, OP), lambda i: (0, 0)),
        ],
        out_specs=[
            pl.BlockSpec((RB, OP), lambda i: (i, 0)),
            pl.BlockSpec((RB, OP), lambda i: (i, 0)),
        ],
        out_shape=[
            jax.ShapeDtypeStruct((N, OP), jnp.bfloat16),
            jax.ShapeDtypeStruct((N, OP), jnp.float32),
        ],
    )(s, s, inv, h, wl, wr, b, wl3, wr3, b3)


def _tc_final_body(s0, s1, inv, r, batch, out):
    h3 = (s0[0].astype(jnp.float32) + s1[0].astype(jnp.float32)) \
        * inv[...] + r[...]
    gids = lax.broadcasted_iota(jnp.int32, (N, G), 1)
    onehot = (batch[...] == gids).astype(jnp.float32)
    pooled = lax.dot_general(onehot, h3, (((0,), (0,)), ((), ())),
                             preferred_element_type=jnp.float32)
    cntg = jnp.sum(onehot, axis=0).reshape(G, 1)
    pm = pooled / jnp.maximum(cntg, 1.0)
    col = lax.broadcasted_iota(jnp.int32, (G, OP), 1)
    valid = col < O
    z = jnp.where(valid, pm, -1e30)
    z = z - jnp.max(z, axis=1, keepdims=True)
    e = jnp.where(valid, jnp.exp(z), 0.0)
    out[...] = e / jnp.sum(e, axis=1, keepdims=True)


def _tc_final(s, inv, r, batch2d):
    return pl.pallas_call(
        _tc_final_body,
        grid=(1,),
        in_specs=[
            pl.BlockSpec((1, N, OP), lambda i: (0, 0, 0)),
            pl.BlockSpec((1, N, OP), lambda i: (1, 0, 0)),
            pl.BlockSpec((N, 1), lambda i: (0, 0)),
            pl.BlockSpec((N, OP), lambda i: (0, 0)),
            pl.BlockSpec((N, 1), lambda i: (0, 0)),
        ],
        out_specs=pl.BlockSpec((G, OP), lambda i: (0, 0)),
        out_shape=jax.ShapeDtypeStruct((G, OP), jnp.float32),
    )(s, s, inv, r, batch2d)


def kernel(x, edge_index, batch, W1l, W1r, b1, W2l, W2r, b2, W3l, W3r, b3):
    ei = jnp.ravel(edge_index.astype(jnp.int32))

    S1, C = _sc_aggr_counts(x.astype(jnp.bfloat16), ei, D)
    h1b, inv = _tc_layer1(S1, C, x, W1l, W1r, b1.reshape(1, H))

    wl3 = jnp.pad(W3l, ((0, 0), (0, OP - O)))
    wr3 = jnp.pad(W3r, ((0, 0), (0, OP - O)))
    b3p = jnp.pad(b3, (0, OP - O)).reshape(1, OP)
    S2 = _sc_aggr(h1b, ei, H)
    p, r = _tc_layer2(S2, inv, h1b, W2l, W2r, b2.reshape(1, H),
                      wl3, wr3, b3p)
    S3 = _sc_aggr(p, ei, OP)

    out = _tc_final(S3, inv, r, batch.astype(jnp.int32).reshape(N, 1))
    return out[:, :O]


# speedup vs baseline: 1.0304x; 1.0153x over previous
import functools

import jax
import jax.numpy as jnp
from jax import lax
from jax.experimental import pallas as pl
from jax.experimental.pallas import tpu as pltpu
from jax.experimental.pallas import tpu_sc as plsc

N = 10000
E = 320000
D = 128
H = 128
O = 40
OP = 48
G = 64
CW = 16

NC = 2
NS = 16
NW = NC * NS
EPW = E // NW
K = 80
NCHUNK = EPW // K
NPAD = 10240
RPS = NPAD // NS
SR = 128


NB = 4

_PARAMS = pltpu.CompilerParams(use_tc_tiling_on_sc=False)
_MESH = dict(core_axis_name="c", subcore_axis_name="s")


def _sc_aggregate(width, dtype, with_counts):
    mesh = plsc.VectorSubcoreMesh(**_MESH)
    NB_ = NB if width >= 128 else 2 * NB
    NI = 2 * NB_
    MAIN = (NCHUNK // NI) * NI

    out_type = [jax.ShapeDtypeStruct((NC, NPAD, width), dtype)]
    scratch = [
        pltpu.VMEM_SHARED((NPAD, width), dtype),
        [pltpu.VMEM((K,), jnp.int32)] * NI,
        [pltpu.VMEM((1, K), jnp.int32)] * NI,
        [pltpu.SemaphoreType.DMA] * NI,
        [pltpu.VMEM((K, width), dtype)] * NB_,
        [pltpu.SemaphoreType.DMA] * NB_,
    ]
    if with_counts:
        out_type.append(jax.ShapeDtypeStruct((NC, NPAD, CW), jnp.float32))
        scratch += [
            pltpu.VMEM_SHARED((NPAD, CW), jnp.float32),
            pltpu.VMEM((K, CW), jnp.float32),
        ]

    def body(*refs):
        if with_counts:
            (feat_hbm, ei_hbm, zw_hbm, zc_hbm, ones_hbm, s_out, c_out,
             acc, sidx, didx, isem, rows, gsem, cacc, ones_v) = refs
        else:
            (feat_hbm, ei_hbm, zw_hbm, s_out,
             acc, sidx, didx, isem, rows, gsem) = refs

        cid = lax.axis_index("c")
        sid = lax.axis_index("s")
        wid = cid * NS + sid
        ebase = wid * EPW
        r0 = sid * RPS

        def idx_load(j, i):
            off = ebase + j * K
            pltpu.async_copy(ei_hbm.at[pl.ds(off, K)], sidx[i], isem[i])
            pltpu.async_copy(ei_hbm.at[pl.ds(E + off, K)], didx[i].at[0],
                             isem[i])

        def idx_wait(j, i):
            off = ebase + j * K
            pltpu.make_async_copy(ei_hbm.at[pl.ds(off, K)], sidx[i],
                                  isem[i]).wait()
            pltpu.make_async_copy(ei_hbm.at[pl.ds(E + off, K)],
                                  didx[i].at[0], isem[i]).wait()

        def gather_start(j, i, b):
            pltpu.async_copy(feat_hbm.at[sidx[i]], rows[b], gsem[b])

        def gather_wait(i, b):
            pltpu.make_async_copy(feat_hbm.at[sidx[i]], rows[b],
                                  gsem[b]).wait()

        for i in range(NI):
            idx_load(i, i)
        pltpu.sync_copy(zw_hbm, rows[0])
        for kk in range(RPS // K):
            pltpu.sync_copy(rows[0], acc.at[pl.ds(r0 + kk * K, K)])
        if with_counts:
            pltpu.sync_copy(zc_hbm, ones_v)
            for kk in range(RPS // K):
                pltpu.sync_copy(ones_v, cacc.at[pl.ds(r0 + kk * K, K)])
            pltpu.sync_copy(ones_hbm, ones_v)
        plsc.subcore_barrier()
        for b in range(NB_):
            idx_wait(b, b)
            gather_start(b, b, b)

        def process(j, i, b):
            gather_wait(i, b)
            pltpu.sync_copy(rows[b], acc.at[didx[i].at[0]], add=True)
            if with_counts:
                pltpu.sync_copy(ones_v, cacc.at[didx[i].at[0]], add=True)

            @pl.when(j + NI < NCHUNK)
            def _():
                idx_load(j + NI, i)

            @pl.when(j + NB_ < NCHUNK)
            def _():
                idx_wait(j + NB_, (i + NB_) % NI)
                gather_start(j + NB_, (i + NB_) % NI, b)

        @pl.loop(0, MAIN, step=NI)
        def _(j0):
            for t in range(NI):
                process(j0 + t, t, t % NB_)

        for j in range(MAIN, NCHUNK):
            process(j, j % NI, j % NB_)

        plsc.subcore_barrier()

        pltpu.sync_copy(acc.at[pl.ds(r0, RPS)],
                        s_out.at[cid, pl.ds(r0, RPS)])
        if with_counts:
            pltpu.sync_copy(cacc.at[pl.ds(r0, RPS)],
                            c_out.at[cid, pl.ds(r0, RPS)])

    return pl.kernel(body, out_type=out_type, mesh=mesh,
                     scratch_types=scratch, compiler_params=_PARAMS)


def _sc_aggr(h, ei, width):
    zw = jnp.zeros((K, width), h.dtype)
    return _sc_aggregate(width, h.dtype, False)(h, ei, zw)[0]


def _sc_aggr_counts(h, ei, width):
    zw = jnp.zeros((K, width), h.dtype)
    zc = jnp.zeros((K, CW), jnp.float32)
    ones = jnp.ones((K, CW), jnp.float32)
    return _sc_aggregate(width, h.dtype, True)(h, ei, zw, zc, ones)


RB = 2000


def _tc_layer1_body(s0, s1, c0, c1, x, wl, wr, b, hb_out, inv_out):
    cnt = c0[0][:, :1] + c1[0][:, :1]
    inv = 1.0 / jnp.maximum(cnt, 1.0)
    aggr = (s0[0].astype(jnp.float32) + s1[0].astype(jnp.float32)) * inv
    acc = jnp.dot(aggr, wl[...], preferred_element_type=jnp.float32)
    acc += jnp.dot(x[...], wr[...], preferred_element_type=jnp.float32)
    h = jnp.maximum(acc + b[...], 0.0)
    hb_out[...] = h.astype(jnp.bfloat16)
    inv_out[...] = inv


def _tc_layer1(s, c, x, wl, wr, b):
    grid = N // RB
    return pl.pallas_call(
        _tc_layer1_body,
        grid=(grid,),
        in_specs=[
            pl.BlockSpec((1, RB, D), lambda i: (0, i, 0)),
            pl.BlockSpec((1, RB, D), lambda i: (1, i, 0)),
            pl.BlockSpec((1, RB, CW), lambda i: (0, i, 0)),
            pl.BlockSpec((1, RB, CW), lambda i: (1, i, 0)),
            pl.BlockSpec((RB, D), lambda i: (i, 0)),
            pl.BlockSpec((D, H), lambda i: (0, 0)),
            pl.BlockSpec((D, H), lambda i: (0, 0)),
            pl.BlockSpec((1, H), lambda i: (0, 0)),
        ],
        out_specs=[
            pl.BlockSpec((RB, H), lambda i: (i, 0)),
            pl.BlockSpec((RB, 1), lambda i: (i, 0)),
        ],
        out_shape=[
            jax.ShapeDtypeStruct((N, H), jnp.bfloat16),
            jax.ShapeDtypeStruct((N, 1), jnp.float32),
        ],
    )(s, s, c, c, x, wl, wr, b)


def _tc_layer2_body(s0, s1, inv, h, wl, wr, b, wl3, wr3, b3,
                    p_out, r_out):
    aggr = (s0[0].astype(jnp.float32) + s1[0].astype(jnp.float32)) \
        * inv[...]
    acc = jnp.dot(aggr, wl[...], preferred_element_type=jnp.float32)
    acc += jnp.dot(h[...].astype(jnp.float32), wr[...],
                   preferred_element_type=jnp.float32)
    h2 = jnp.maximum(acc + b[...], 0.0)
    p = jnp.dot(h2, wl3[...], preferred_element_type=jnp.float32)
    p_out[...] = p.astype(jnp.bfloat16)
    r_out[...] = jnp.dot(h2, wr3[...],
                         preferred_element_type=jnp.float32) + b3[...]


def _tc_layer2(s, inv, h, wl, wr, b, wl3, wr3, b3):
    grid = N // RB
    return pl.pallas_call(
        _tc_layer2_body,
        grid=(grid,),
        in_specs=[
            pl.BlockSpec((1, RB, H), lambda i: (0, i, 0)),
            pl.BlockSpec((1, RB, H), lambda i: (1, i, 0)),
            pl.BlockSpec((RB, 1), lambda i: (i, 0)),
            pl.BlockSpec((RB, H), lambda i: (i, 0)),
            pl.BlockSpec((H, H), lambda i: (0, 0)),
            pl.BlockSpec((H, H), lambda i: (0, 0)),
            pl.BlockSpec((1, H), lambda i: (0, 0)),
            pl.BlockSpec((H, OP), lambda i: (0, 0)),
            pl.BlockSpec((H, OP), lambda i: (0, 0)),
            pl.BlockSpec((1, OP), lambda i: (0, 0)),
        ],
        out_specs=[
            pl.BlockSpec((RB, OP), lambda i: (i, 0)),
            pl.BlockSpec((RB, OP), lambda i: (i, 0)),
        ],
        out_shape=[
            jax.ShapeDtypeStruct((N, OP), jnp.bfloat16),
            jax.ShapeDtypeStruct((N, OP), jnp.float32),
        ],
    )(s, s, inv, h, wl, wr, b, wl3, wr3, b3)


def _tc_final_body(s0, s1, inv, r, batch, out):
    h3 = (s0[0].astype(jnp.float32) + s1[0].astype(jnp.float32)) \
        * inv[...] + r[...]
    gids = lax.broadcasted_iota(jnp.int32, (N, G), 1)
    onehot = (batch[...] == gids).astype(jnp.float32)
    pooled = lax.dot_general(onehot, h3, (((0,), (0,)), ((), ())),
                             preferred_element_type=jnp.float32)
    cntg = jnp.sum(onehot, axis=0).reshape(G, 1)
    pm = pooled / jnp.maximum(cntg, 1.0)
    col = lax.broadcasted_iota(jnp.int32, (G, OP), 1)
    valid = col < O
    z = jnp.where(valid, pm, -1e30)
    z = z - jnp.max(z, axis=1, keepdims=True)
    e = jnp.where(valid, jnp.exp(z), 0.0)
    out[...] = e / jnp.sum(e, axis=1, keepdims=True)


def _tc_final(s, inv, r, batch2d):
    return pl.pallas_call(
        _tc_final_body,
        grid=(1,),
        in_specs=[
            pl.BlockSpec((1, N, OP), lambda i: (0, 0, 0)),
            pl.BlockSpec((1, N, OP), lambda i: (1, 0, 0)),
            pl.BlockSpec((N, 1), lambda i: (0, 0)),
            pl.BlockSpec((N, OP), lambda i: (0, 0)),
            pl.BlockSpec((N, 1), lambda i: (0, 0)),
        ],
        out_specs=pl.BlockSpec((G, OP), lambda i: (0, 0)),
        out_shape=jax.ShapeDtypeStruct((G, OP), jnp.float32),
    )(s, s, inv, r, batch2d)


def kernel(x, edge_index, batch, W1l, W1r, b1, W2l, W2r, b2, W3l, W3r, b3):
    ei = jnp.ravel(edge_index.astype(jnp.int32))

    S1, C = _sc_aggr_counts(x.astype(jnp.bfloat16), ei, D)
    h1b, inv = _tc_layer1(S1, C, x, W1l, W1r, b1.reshape(1, H))

    wl3 = jnp.pad(W3l, ((0, 0), (0, OP - O)))
    wr3 = jnp.pad(W3r, ((0, 0), (0, OP - O)))
    b3p = jnp.pad(b3, (0, OP - O)).reshape(1, OP)
    S2 = _sc_aggr(h1b, ei, H)
    p, r = _tc_layer2(S2, inv, h1b, W2l, W2r, b2.reshape(1, H),
                      wl3, wr3, b3p)
    S3 = _sc_aggr(p, ei, OP)

    out = _tc_final(S3, inv, r, batch.astype(jnp.int32).reshape(N, 1))
    return out[:, :O]
